# baseline (device time: 112550 ns/iter reference)
import jax
import jax.numpy as jnp
from jax import lax
from jax.experimental import pallas as pl
from jax.experimental.pallas import tpu as pltpu

NDEV = 8
B, SQ, SKV = 2, 512, 512
HL = 8
DH = 64
DM = 768
DQ = HL * DH
ROWS = B * SQ
CHUNK = ROWS // NDEV
BLK = 64


def kernel(x, Wq, K_ext, V_ext, Wo):
    my = lax.axis_index("i")
    Wq_s = lax.dynamic_slice(Wq, (0, my * DQ), (DM, DQ)).astype(jnp.bfloat16)
    Wo_s = lax.dynamic_slice(Wo, (my * DQ, 0), (DQ, DM)).astype(jnp.bfloat16)

    def body(x_ref, wq_ref, k_ref, v_ref, wo_ref, out_ref,
             acc_ref, rbuf_ref, send_sems, recv_sems):
        my_pos = lax.axis_index("i")
        left = (my_pos - 1) % NDEV
        right = (my_pos + 1) % NDEV

        barrier_sem = pltpu.get_barrier_semaphore()
        for nbr in (left, right):
            pl.semaphore_signal(
                barrier_sem, inc=1,
                device_id=(nbr,), device_id_type=pl.DeviceIdType.MESH,
            )
        pl.semaphore_wait(barrier_sem, 2)

        row_blk = lax.broadcasted_iota(jnp.int32, (SQ, SKV), 0) // BLK
        col_blk = lax.broadcasted_iota(jnp.int32, (SQ, SKV), 1) // BLK
        mask = col_blk <= row_blk

        wq = wq_ref[...]
        wo = wo_ref[...]
        for b in range(B):
            xb = x_ref[b].astype(jnp.bfloat16)
            qb = lax.dot(xb, wq, preferred_element_type=jnp.float32)
            ctx_heads = []
            for h in range(HL):
                qh = qb[:, h * DH:(h + 1) * DH].astype(jnp.bfloat16)
                kh = k_ref[b, :, h, :].astype(jnp.bfloat16)
                vh = v_ref[b, :, h, :].astype(jnp.bfloat16)
                s = lax.dot_general(
                    qh, kh, (((1,), (1,)), ((), ())),
                    preferred_element_type=jnp.float32,
                ) * 0.125
                s = jnp.where(mask, s, -1e9)
                m = jnp.max(s, axis=-1, keepdims=True)
                w = jnp.exp(s - m)
                w = w / jnp.sum(w, axis=-1, keepdims=True)
                ctx_heads.append(
                    lax.dot(w.astype(jnp.bfloat16), vh,
                            preferred_element_type=jnp.float32)
                )
            ctx = jnp.concatenate(ctx_heads, axis=-1).astype(jnp.bfloat16)
            acc_ref[b * SQ:(b + 1) * SQ, :] = lax.dot(
                ctx, wo, preferred_element_type=jnp.float32)

        for s_ in range(NDEV - 1):
            c_send = (my_pos - s_) % NDEV
            c_recv = (my_pos - s_ - 1) % NDEV
            rdma = pltpu.make_async_remote_copy(
                src_ref=acc_ref.at[pl.ds(c_send * CHUNK, CHUNK), :],
                dst_ref=rbuf_ref.at[s_],
                send_sem=send_sems.at[s_],
                recv_sem=recv_sems.at[s_],
                device_id=(right,),
                device_id_type=pl.DeviceIdType.MESH,
            )
            rdma.start()
            rdma.wait()
            acc_ref[pl.ds(c_recv * CHUNK, CHUNK), :] = (
                acc_ref[pl.ds(c_recv * CHUNK, CHUNK), :] + rbuf_ref[s_]
            )

        for t in range(NDEV - 1):
            c_fwd = (my_pos + 1 - t) % NDEV
            rdma = pltpu.make_async_remote_copy(
                src_ref=acc_ref.at[pl.ds(c_fwd * CHUNK, CHUNK), :],
                dst_ref=acc_ref.at[pl.ds(c_fwd * CHUNK, CHUNK), :],
                send_sem=send_sems.at[NDEV - 1 + t],
                recv_sem=recv_sems.at[NDEV - 1 + t],
                device_id=(right,),
                device_id_type=pl.DeviceIdType.MESH,
            )
            rdma.start()
            rdma.wait()

        for b in range(B):
            out_ref[b, :, :] = acc_ref[b * SQ:(b + 1) * SQ, :]

    return pl.pallas_call(
        body,
        out_shape=jax.ShapeDtypeStruct((B, SQ, DM), jnp.float32),
        in_specs=[pl.BlockSpec(memory_space=pltpu.VMEM)] * 5,
        out_specs=pl.BlockSpec(memory_space=pltpu.VMEM),
        scratch_shapes=[
            pltpu.VMEM((ROWS, DM), jnp.float32),
            pltpu.VMEM((NDEV - 1, CHUNK, DM), jnp.float32),
            pltpu.SemaphoreType.DMA((2 * (NDEV - 1),)),
            pltpu.SemaphoreType.DMA((2 * (NDEV - 1),)),
        ],
        compiler_params=pltpu.CompilerParams(collective_id=0),
    )(x, Wq_s, K_ext, V_ext, Wo_s)


# device time: 53476 ns/iter; 2.1047x vs baseline; 2.1047x over previous
import jax
import jax.numpy as jnp
from jax import lax
from jax.experimental import pallas as pl
from jax.experimental.pallas import tpu as pltpu

NDEV = 8
B, SQ, SKV = 2, 512, 512
HL = 8
DH = 64
DM = 768
DQ = HL * DH
ROWS = B * SQ
CHUNK = ROWS // NDEV
BLK = 64


def kernel(x, Wq, K_ext, V_ext, Wo):
    my = lax.axis_index("i")
    Wq_s = lax.dynamic_slice(Wq, (0, my * DQ), (DM, DQ)).astype(jnp.bfloat16)
    Wo_s = lax.dynamic_slice(Wo, (my * DQ, 0), (DQ, DM)).astype(jnp.bfloat16)

    def body(x_ref, wq_ref, k_ref, v_ref, wo_ref, out_ref,
             sbuf, rbuf, abuf, gbuf, ss_rs, rs_sems, ss_ag, ag_sems):
        my_pos = lax.axis_index("i")

        barrier_sem = pltpu.get_barrier_semaphore()
        for o in range(1, NDEV):
            pl.semaphore_signal(
                barrier_sem, inc=1,
                device_id=((my_pos + o) % NDEV,),
                device_id_type=pl.DeviceIdType.MESH,
            )
        pl.semaphore_wait(barrier_sem, NDEV - 1)

        row_blk = lax.broadcasted_iota(jnp.int32, (SQ, SKV), 0) // BLK
        col_blk = lax.broadcasted_iota(jnp.int32, (SQ, SKV), 1) // BLK
        mask = col_blk <= row_blk

        wq = wq_ref[...]
        wo = wo_ref[...]
        for b in range(B):
            xb = x_ref[b].astype(jnp.bfloat16)
            qb = lax.dot(xb, wq, preferred_element_type=jnp.float32)
            ctx_heads = []
            for h in range(HL):
                qh = qb[:, h * DH:(h + 1) * DH].astype(jnp.bfloat16)
                kh = k_ref[b, :, h, :].astype(jnp.bfloat16)
                vh = v_ref[b, :, h, :].astype(jnp.bfloat16)
                s = lax.dot_general(
                    qh, kh, (((1,), (1,)), ((), ())),
                    preferred_element_type=jnp.float32,
                ) * 0.125
                s = jnp.where(mask, s, -1e9)
                m = jnp.max(s, axis=-1, keepdims=True)
                w = jnp.exp(s - m)
                w = w / jnp.sum(w, axis=-1, keepdims=True)
                ctx_heads.append(
                    lax.dot(w.astype(jnp.bfloat16), vh,
                            preferred_element_type=jnp.float32)
                )
            ctx = jnp.concatenate(ctx_heads, axis=-1).astype(jnp.bfloat16)
            sbuf[b * SQ:(b + 1) * SQ, :] = lax.dot(
                ctx, wo, preferred_element_type=jnp.float32).astype(jnp.bfloat16)

        rs_sends = []
        for o in range(1, NDEV):
            t = (my_pos + o) % NDEV
            rdma = pltpu.make_async_remote_copy(
                src_ref=sbuf.at[pl.ds(t * CHUNK, CHUNK), :],
                dst_ref=rbuf.at[pl.ds((NDEV - 1 - o) * CHUNK, CHUNK), :],
                send_sem=ss_rs.at[o - 1],
                recv_sem=rs_sems.at[NDEV - 1 - o],
                device_id=(t,),
                device_id_type=pl.DeviceIdType.MESH,
            )
            rdma.start()
            rs_sends.append(rdma)

        red = sbuf[pl.ds(my_pos * CHUNK, CHUNK), :].astype(jnp.float32)
        for o, rdma in zip(range(1, NDEV), rs_sends):
            rdma.wait()
            j = NDEV - 1 - o
            red = red + rbuf[pl.ds(j * CHUNK, CHUNK), :].astype(jnp.float32)

        sbuf[pl.ds(my_pos * CHUNK, CHUNK), :] = red.astype(jnp.bfloat16)

        ag_sends = []
        for o in range(1, NDEV):
            t = (my_pos + o) % NDEV
            rdma = pltpu.make_async_remote_copy(
                src_ref=sbuf.at[pl.ds(my_pos * CHUNK, CHUNK), :],
                dst_ref=abuf.at[pl.ds((NDEV - 1 - o) * CHUNK, CHUNK), :],
                send_sem=ss_ag.at[o - 1],
                recv_sem=ag_sems.at[NDEV - 1 - o],
                device_id=(t,),
                device_id_type=pl.DeviceIdType.MESH,
            )
            rdma.start()
            ag_sends.append(rdma)

        gbuf[pl.ds(my_pos * CHUNK, CHUNK), :] = red.astype(jnp.bfloat16)
        for o, rdma in zip(range(1, NDEV), ag_sends):
            rdma.wait()
            j = NDEV - 1 - o
            c = (my_pos - o) % NDEV
            gbuf[pl.ds(c * CHUNK, CHUNK), :] = abuf[pl.ds(j * CHUNK, CHUNK), :]

        for b in range(B):
            out_ref[b, :, :] = gbuf[b * SQ:(b + 1) * SQ, :].astype(jnp.float32)

    return pl.pallas_call(
        body,
        out_shape=jax.ShapeDtypeStruct((B, SQ, DM), jnp.float32),
        in_specs=[pl.BlockSpec(memory_space=pltpu.VMEM)] * 5,
        out_specs=pl.BlockSpec(memory_space=pltpu.VMEM),
        scratch_shapes=[
            pltpu.VMEM((ROWS, DM), jnp.bfloat16),
            pltpu.VMEM(((NDEV - 1) * CHUNK, DM), jnp.bfloat16),
            pltpu.VMEM(((NDEV - 1) * CHUNK, DM), jnp.bfloat16),
            pltpu.VMEM((ROWS, DM), jnp.bfloat16),
            pltpu.SemaphoreType.DMA((NDEV - 1,)),
            pltpu.SemaphoreType.DMA((NDEV - 1,)),
            pltpu.SemaphoreType.DMA((NDEV - 1,)),
            pltpu.SemaphoreType.DMA((NDEV - 1,)),
        ],
        compiler_params=pltpu.CompilerParams(collective_id=0),
    )(x, Wq_s, K_ext, V_ext, Wo_s)


# device time: 52712 ns/iter; 2.1352x vs baseline; 1.0145x over previous
import jax
import jax.numpy as jnp
from jax import lax
from jax.experimental import pallas as pl
from jax.experimental.pallas import tpu as pltpu

NDEV = 8
B, SQ, SKV = 2, 512, 512
HL = 8
DH = 64
DM = 768
DQ = HL * DH
ROWS = B * SQ
CHUNK = ROWS // NDEV
BLK = 64


def kernel(x, Wq, K_ext, V_ext, Wo):
    my = lax.axis_index("i")
    Wq_s = lax.dynamic_slice(Wq, (0, my * DQ), (DM, DQ)).astype(jnp.bfloat16)
    Wo_s = lax.dynamic_slice(Wo, (my * DQ, 0), (DQ, DM)).astype(jnp.bfloat16)

    def body(x_ref, wq_ref, k_ref, v_ref, wo_ref, out_ref,
             sbuf, rbuf, ss_rs, rs_sems, ss_ag, ag_sems):
        my_pos = lax.axis_index("i")

        barrier_sem = pltpu.get_barrier_semaphore()
        for o in range(1, NDEV):
            pl.semaphore_signal(
                barrier_sem, inc=1,
                device_id=((my_pos + o) % NDEV,),
                device_id_type=pl.DeviceIdType.MESH,
            )
        pl.semaphore_wait(barrier_sem, NDEV - 1)

        row_blk = lax.broadcasted_iota(jnp.int32, (SQ, SKV), 0) // BLK
        col_blk = lax.broadcasted_iota(jnp.int32, (SQ, SKV), 1) // BLK
        mask = col_blk <= row_blk

        wq = wq_ref[...]
        wo = wo_ref[...]
        for b in range(B):
            xb = x_ref[b].astype(jnp.bfloat16)
            qb = lax.dot(xb, wq, preferred_element_type=jnp.float32)
            ctx_heads = []
            for h in range(HL):
                qh = qb[:, h * DH:(h + 1) * DH].astype(jnp.bfloat16)
                kh = k_ref[b, :, h, :].astype(jnp.bfloat16)
                vh = v_ref[b, :, h, :].astype(jnp.bfloat16)
                s = lax.dot_general(
                    qh, kh, (((1,), (1,)), ((), ())),
                    preferred_element_type=jnp.float32,
                ) * 0.125
                s = jnp.where(mask, s, -1e9)
                m = jnp.max(s, axis=-1, keepdims=True)
                w = jnp.exp(s - m)
                w = w / jnp.sum(w, axis=-1, keepdims=True)
                ctx_heads.append(
                    lax.dot(w.astype(jnp.bfloat16), vh,
                            preferred_element_type=jnp.float32)
                )
            ctx = jnp.concatenate(ctx_heads, axis=-1).astype(jnp.bfloat16)
            sbuf[b * SQ:(b + 1) * SQ, :] = lax.dot(
                ctx, wo, preferred_element_type=jnp.float32).astype(jnp.bfloat16)

        rs_sends = []
        for o in range(1, NDEV):
            t = (my_pos + o) % NDEV
            rdma = pltpu.make_async_remote_copy(
                src_ref=sbuf.at[pl.ds(t * CHUNK, CHUNK), :],
                dst_ref=rbuf.at[pl.ds((NDEV - 1 - o) * CHUNK, CHUNK), :],
                send_sem=ss_rs.at[o - 1],
                recv_sem=rs_sems.at[NDEV - 1 - o],
                device_id=(t,),
                device_id_type=pl.DeviceIdType.MESH,
            )
            rdma.start()
            rs_sends.append(rdma)

        red = sbuf[pl.ds(my_pos * CHUNK, CHUNK), :].astype(jnp.float32)
        for o, rdma in zip(range(1, NDEV), rs_sends):
            rdma.wait()
            j = NDEV - 1 - o
            red = red + rbuf[pl.ds(j * CHUNK, CHUNK), :].astype(jnp.float32)

        red16 = red.astype(jnp.bfloat16)
        sbuf[pl.ds(my_pos * CHUNK, CHUNK), :] = red16
        out_ref[pl.ds(my_pos * CHUNK, CHUNK), :] = red16

        ag_sends = []
        for o in range(1, NDEV):
            t = (my_pos + o) % NDEV
            rdma = pltpu.make_async_remote_copy(
                src_ref=sbuf.at[pl.ds(my_pos * CHUNK, CHUNK), :],
                dst_ref=out_ref.at[pl.ds(my_pos * CHUNK, CHUNK), :],
                send_sem=ss_ag.at[o - 1],
                recv_sem=ag_sems.at[NDEV - 1 - o],
                device_id=(t,),
                device_id_type=pl.DeviceIdType.MESH,
            )
            rdma.start()
            ag_sends.append(rdma)
        for rdma in ag_sends:
            rdma.wait()

    out = pl.pallas_call(
        body,
        out_shape=jax.ShapeDtypeStruct((ROWS, DM), jnp.bfloat16),
        in_specs=[pl.BlockSpec(memory_space=pltpu.VMEM)] * 5,
        out_specs=pl.BlockSpec(memory_space=pltpu.VMEM),
        scratch_shapes=[
            pltpu.VMEM((ROWS, DM), jnp.bfloat16),
            pltpu.VMEM(((NDEV - 1) * CHUNK, DM), jnp.bfloat16),
            pltpu.SemaphoreType.DMA((NDEV - 1,)),
            pltpu.SemaphoreType.DMA((NDEV - 1,)),
            pltpu.SemaphoreType.DMA((NDEV - 1,)),
            pltpu.SemaphoreType.DMA((NDEV - 1,)),
        ],
        compiler_params=pltpu.CompilerParams(collective_id=0),
    )(x, Wq_s, K_ext, V_ext, Wo_s)
    return out.reshape(B, SQ, DM)


# device time: 47251 ns/iter; 2.3820x vs baseline; 1.1156x over previous
import jax
import jax.numpy as jnp
from jax import lax
from jax.experimental import pallas as pl
from jax.experimental.pallas import tpu as pltpu

NDEV = 8
B, SQ, SKV = 2, 512, 512
HL = 8
DH = 64
DM = 768
DQ = HL * DH
ROWS = B * SQ
CHUNK = ROWS // NDEV
BLK = 64


def kernel(x, Wq, K_ext, V_ext, Wo):
    my = lax.axis_index("i")
    Wq_s = lax.dynamic_slice(Wq, (0, my * DQ), (DM, DQ)).astype(jnp.bfloat16)
    Wo_s = lax.dynamic_slice(Wo, (my * DQ, 0), (DQ, DM)).astype(jnp.bfloat16)

    def body(x_ref, wq_ref, k_ref, v_ref, wo_ref, out_ref,
             sbuf, rbuf, ss_rs, rs_sems, ss_ag, ag_sems):
        my_pos = lax.axis_index("i")

        barrier_sem = pltpu.get_barrier_semaphore()
        for o in range(1, NDEV):
            pl.semaphore_signal(
                barrier_sem, inc=1,
                device_id=((my_pos + o) % NDEV,),
                device_id_type=pl.DeviceIdType.MESH,
            )
        pl.semaphore_wait(barrier_sem, NDEV - 1)

        row_blk = lax.broadcasted_iota(jnp.int32, (SQ, SKV), 0) // BLK
        col_blk = lax.broadcasted_iota(jnp.int32, (SQ, SKV), 1) // BLK
        mask = col_blk <= row_blk

        rs_sends = []
        for o in range(1, NDEV):
            t = (my_pos + o) % NDEV
            rdma = pltpu.make_async_remote_copy(
                src_ref=sbuf.at[pl.ds(t * CHUNK, CHUNK), :],
                dst_ref=rbuf.at[pl.ds((NDEV - 1 - o) * CHUNK, CHUNK), :],
                send_sem=ss_rs.at[o - 1],
                recv_sem=rs_sems.at[NDEV - 1 - o],
                device_id=(t,),
                device_id_type=pl.DeviceIdType.MESH,
            )
            rs_sends.append((t, rdma))

        wq = wq_ref[...]
        wo = wo_ref[...]
        chunks_per_b = SQ // CHUNK
        for b in range(B):
            xb = x_ref[b].astype(jnp.bfloat16)
            qb = lax.dot(xb, wq, preferred_element_type=jnp.float32)
            ctx_heads = []
            for h in range(HL):
                qh = qb[:, h * DH:(h + 1) * DH].astype(jnp.bfloat16)
                kh = k_ref[b, :, h, :].astype(jnp.bfloat16)
                vh = v_ref[b, :, h, :].astype(jnp.bfloat16)
                s = lax.dot_general(
                    qh, kh, (((1,), (1,)), ((), ())),
                    preferred_element_type=jnp.float32,
                ) * 0.125
                s = jnp.where(mask, s, -1e9)
                m = jnp.max(s, axis=-1, keepdims=True)
                w = jnp.exp(s - m)
                w = w / jnp.sum(w, axis=-1, keepdims=True)
                ctx_heads.append(
                    lax.dot(w.astype(jnp.bfloat16), vh,
                            preferred_element_type=jnp.float32)
                )
            ctx = jnp.concatenate(ctx_heads, axis=-1).astype(jnp.bfloat16)
            sbuf[b * SQ:(b + 1) * SQ, :] = lax.dot(
                ctx, wo, preferred_element_type=jnp.float32).astype(jnp.bfloat16)
            lo, hi = b * chunks_per_b, (b + 1) * chunks_per_b
            for t, rdma in rs_sends:
                @pl.when(jnp.logical_and(t >= lo, t < hi))
                def _():
                    rdma.start()

        red = sbuf[pl.ds(my_pos * CHUNK, CHUNK), :].astype(jnp.float32)
        for o, (_, rdma) in zip(range(1, NDEV), rs_sends):
            rdma.wait()
            j = NDEV - 1 - o
            red = red + rbuf[pl.ds(j * CHUNK, CHUNK), :].astype(jnp.float32)

        red16 = red.astype(jnp.bfloat16)
        sbuf[pl.ds(my_pos * CHUNK, CHUNK), :] = red16
        out_ref[pl.ds(my_pos * CHUNK, CHUNK), :] = red16

        ag_sends = []
        for o in range(1, NDEV):
            t = (my_pos + o) % NDEV
            rdma = pltpu.make_async_remote_copy(
                src_ref=sbuf.at[pl.ds(my_pos * CHUNK, CHUNK), :],
                dst_ref=out_ref.at[pl.ds(my_pos * CHUNK, CHUNK), :],
                send_sem=ss_ag.at[o - 1],
                recv_sem=ag_sems.at[NDEV - 1 - o],
                device_id=(t,),
                device_id_type=pl.DeviceIdType.MESH,
            )
            rdma.start()
            ag_sends.append(rdma)
        for rdma in ag_sends:
            rdma.wait()

    out = pl.pallas_call(
        body,
        out_shape=jax.ShapeDtypeStruct((ROWS, DM), jnp.bfloat16),
        in_specs=[pl.BlockSpec(memory_space=pltpu.VMEM)] * 5,
        out_specs=pl.BlockSpec(memory_space=pltpu.VMEM),
        scratch_shapes=[
            pltpu.VMEM((ROWS, DM), jnp.bfloat16),
            pltpu.VMEM(((NDEV - 1) * CHUNK, DM), jnp.bfloat16),
            pltpu.SemaphoreType.DMA((NDEV - 1,)),
            pltpu.SemaphoreType.DMA((NDEV - 1,)),
            pltpu.SemaphoreType.DMA((NDEV - 1,)),
            pltpu.SemaphoreType.DMA((NDEV - 1,)),
        ],
        compiler_params=pltpu.CompilerParams(collective_id=0),
    )(x, Wq_s, K_ext, V_ext, Wo_s)
    return out.reshape(B, SQ, DM)
